# baseline (speedup 1.0000x reference)
import functools

import jax
import jax.numpy as jnp
from jax import lax
from jax.experimental import pallas as pl
from jax.experimental.pallas import tpu as pltpu
from jax.experimental.pallas import tpu_sc as plsc

_N = 10000
_E = 320000
_H = 32

_NC = 2
_NS = 16
_NW = _NC * _NS
_CH = 128
_R = 1
_K = 80
_EPAD = _NW * _K * _CH
_NPAD = 10240
_RPT = _NPAD // _NS

_mesh = plsc.VectorSubcoreMesh(core_axis_name="c", subcore_axis_name="s")


@functools.partial(
    pl.kernel,
    out_type=jax.ShapeDtypeStruct((_NC, _NPAD), jnp.float32),
    mesh=_mesh,
    scratch_types=[
        pltpu.VMEM((_K, _CH), jnp.int32),
        pltpu.VMEM((_CH,), jnp.float32),
        pltpu.VMEM_SHARED((_NPAD,), jnp.float32),
    ],
)
def _sc_degree(dst_hbm, zero_hbm, out_hbm, dst_v, ones_v, acc_sh):
    c = lax.axis_index("c")
    s = lax.axis_index("s")
    wid = s * _NC + c
    pltpu.sync_copy(dst_hbm.at[wid], dst_v)
    for i in range(_CH // 16):
        ones_v[pl.ds(i * 16, 16)] = jnp.ones((16,), jnp.float32)
    pltpu.sync_copy(zero_hbm.at[pl.ds(s * _RPT, _RPT)],
                    acc_sh.at[pl.ds(s * _RPT, _RPT)])
    plsc.subcore_barrier()

    def body(j, carry):
        pltpu.sync_copy(ones_v, acc_sh.at[dst_v.at[j]], add=True)
        return carry

    lax.fori_loop(0, _K, body, 0)
    plsc.subcore_barrier()
    pltpu.sync_copy(acc_sh.at[pl.ds(s * _RPT, _RPT)],
                    out_hbm.at[c, pl.ds(s * _RPT, _RPT)])


@functools.partial(
    pl.kernel,
    out_type=jax.ShapeDtypeStruct((_NC, _NPAD, _H), jnp.float32),
    mesh=_mesh,
    scratch_types=[
        pltpu.VMEM((_K, _CH), jnp.int32),
        pltpu.VMEM((_K, _CH), jnp.int32),
        pltpu.VMEM((_CH, _H), jnp.float32),
        pltpu.VMEM_SHARED((_NPAD, _H), jnp.float32),
        pltpu.SemaphoreType.DMA,
    ],
    compiler_params=pltpu.CompilerParams(use_tc_tiling_on_sc=False),
)
def _sc_scatter(hs_hbm, src_hbm, dst_hbm, zero_hbm, out_hbm,
                src_v, dst_v, rows_v, acc_sh, gsem):
    c = lax.axis_index("c")
    s = lax.axis_index("s")
    wid = s * _NC + c
    pltpu.sync_copy(src_hbm.at[wid], src_v)
    pltpu.sync_copy(dst_hbm.at[wid], dst_v)
    pltpu.sync_copy(zero_hbm.at[pl.ds(s * _RPT, _RPT)],
                    acc_sh.at[pl.ds(s * _RPT, _RPT)])
    plsc.subcore_barrier()

    def body(j, carry):
        pltpu.async_copy(hs_hbm.at[src_v.at[j]], rows_v, gsem).wait()
        pltpu.sync_copy(rows_v, acc_sh.at[dst_v.at[j]], add=True)
        return carry

    lax.fori_loop(0, _K, body, 0)
    plsc.subcore_barrier()
    pltpu.sync_copy(acc_sh.at[pl.ds(s * _RPT, _RPT)],
                    out_hbm.at[c, pl.ds(s * _RPT, _RPT)])


def _dinv_body(c0_ref, c1_ref, o_ref):
    deg = c0_ref[...] + c1_ref[...] + 1.0
    o_ref[...] = lax.rsqrt(deg)


def _dense1_body(x_ref, w_ref, dinv_ref, o_ref):
    hw = jnp.dot(x_ref[...], w_ref[...], preferred_element_type=jnp.float32)
    o_ref[...] = dinv_ref[...] * hw


def _combine_body(p0_ref, p1_ref, hs_ref, dinv_ref, b_ref, w_ref, o_ref):
    agg = p0_ref[...] + p1_ref[...] + hs_ref[...]
    h = jnp.maximum(dinv_ref[...] * agg + b_ref[...], 0.0)
    o_ref[...] = dinv_ref[...] * jnp.dot(
        h, w_ref[...], preferred_element_type=jnp.float32)


def _final_body(p0_ref, p1_ref, hs_ref, dinv_ref, b_ref, o_ref):
    h = dinv_ref[...] * (p0_ref[...] + p1_ref[...] + hs_ref[...]) + b_ref[...]
    m = jnp.max(h, axis=1, keepdims=True)
    e = jnp.exp(h - m)
    lse = jnp.log(jnp.sum(e, axis=1, keepdims=True)) + m
    o_ref[...] = h - lse


def _tc(body, out_shape, *ins):
    if isinstance(out_shape, tuple) and isinstance(out_shape[0], tuple):
        os = tuple(jax.ShapeDtypeStruct(s, jnp.float32) for s in out_shape)
    else:
        os = jax.ShapeDtypeStruct(out_shape, jnp.float32)
    return pl.pallas_call(body, out_shape=os)(*ins)


def kernel(x, edge_index, W1, b1, W2, b2, W3, b3):
    src = edge_index[0]
    dst = edge_index[1]
    pad = _EPAD - _E
    src_p = jnp.concatenate(
        [src, jnp.zeros((pad,), jnp.int32)]).reshape(_NW, _K, _CH)
    dst_pad = _N + (jnp.arange(pad, dtype=jnp.int32) % (_NPAD - _N))
    dst_p = jnp.concatenate([dst, dst_pad]).reshape(_NW, _K, _CH)
    zero1 = jnp.zeros((_NPAD,), jnp.float32)
    zero2 = jnp.zeros((_NPAD, _H), jnp.float32)

    cnt = _sc_degree(dst_p, zero1)
    dinv2d = _tc(_dinv_body, (_NPAD // 128, 128),
                 cnt[0].reshape(_NPAD // 128, 128),
                 cnt[1].reshape(_NPAD // 128, 128))
    dinv = dinv2d.reshape(_NPAD)[:_N, None]

    hs1 = _tc(_dense1_body, (_N, _H), x, W1, dinv)
    p = _sc_scatter(hs1, src_p, dst_p, zero2)
    hs2 = _tc(_combine_body, (_N, _H),
              p[0, :_N], p[1, :_N], hs1, dinv, b1[None, :], W2)
    p = _sc_scatter(hs2, src_p, dst_p, zero2)
    hs3 = _tc(_combine_body, (_N, _H),
              p[0, :_N], p[1, :_N], hs2, dinv, b2[None, :], W3)
    p = _sc_scatter(hs3, src_p, dst_p, zero2)
    return _tc(_final_body, (_N, _H),
               p[0, :_N], p[1, :_N], hs3, dinv, b3[None, :])


# speedup vs baseline: 2.0950x; 2.0950x over previous
import functools

import jax
import jax.numpy as jnp
from jax import lax
from jax.experimental import pallas as pl
from jax.experimental.pallas import tpu as pltpu
from jax.experimental.pallas import tpu_sc as plsc

_N = 10000
_E = 320000
_H = 32

_NC = 2
_NS = 16
_NW = _NC * _NS
_CH = 128
_G = 4
_K = 80
_EPAD = _NW * _K * _CH
_NPAD = 10240
_RPT = _NPAD // _NS

_mesh = plsc.VectorSubcoreMesh(core_axis_name="c", subcore_axis_name="s")


@functools.partial(
    pl.kernel,
    out_type=jax.ShapeDtypeStruct((_NC, _NPAD), jnp.float32),
    mesh=_mesh,
    scratch_types=[
        pltpu.VMEM((_K, _CH), jnp.int32),
        pltpu.VMEM((_CH,), jnp.float32),
        pltpu.VMEM_SHARED((_NPAD,), jnp.float32),
    ],
)
def _sc_degree(dst_hbm, zero_hbm, out_hbm, dst_v, ones_v, acc_sh):
    c = lax.axis_index("c")
    s = lax.axis_index("s")
    wid = s * _NC + c
    pltpu.sync_copy(dst_hbm.at[wid], dst_v)
    for i in range(_CH // 16):
        ones_v[pl.ds(i * 16, 16)] = jnp.ones((16,), jnp.float32)
    pltpu.sync_copy(zero_hbm.at[pl.ds(s * _RPT, _RPT)],
                    acc_sh.at[pl.ds(s * _RPT, _RPT)])
    plsc.subcore_barrier()

    def body(j, carry):
        pltpu.sync_copy(ones_v, acc_sh.at[dst_v.at[j]], add=True)
        return carry

    lax.fori_loop(0, _K, body, 0)
    plsc.subcore_barrier()
    pltpu.sync_copy(acc_sh.at[pl.ds(s * _RPT, _RPT)],
                    out_hbm.at[c, pl.ds(s * _RPT, _RPT)])


@functools.partial(
    pl.kernel,
    out_type=jax.ShapeDtypeStruct((_NC, _NPAD, _H), jnp.float32),
    mesh=_mesh,
    scratch_types=[
        pltpu.VMEM((_K, _CH), jnp.int32),
        pltpu.VMEM((_K, _CH), jnp.int32),
        pltpu.VMEM((_CH, _H), jnp.float32),
        pltpu.VMEM((_CH, _H), jnp.float32),
        pltpu.VMEM((_CH, _H), jnp.float32),
        pltpu.VMEM((_CH, _H), jnp.float32),
        pltpu.VMEM_SHARED((_NPAD, _H), jnp.float32),
        pltpu.SemaphoreType.DMA,
        pltpu.SemaphoreType.DMA,
        pltpu.SemaphoreType.DMA,
        pltpu.SemaphoreType.DMA,
    ],
    compiler_params=pltpu.CompilerParams(use_tc_tiling_on_sc=False),
)
def _sc_scatter(hs_hbm, src_hbm, dst_hbm, zero_hbm, out_hbm,
                src_v, dst_v, r0, r1, r2, r3, acc_sh, g0, g1, g2, g3):
    c = lax.axis_index("c")
    s = lax.axis_index("s")
    wid = s * _NC + c
    pltpu.sync_copy(src_hbm.at[wid], src_v)
    pltpu.sync_copy(dst_hbm.at[wid], dst_v)
    pltpu.sync_copy(zero_hbm.at[pl.ds(s * _RPT, _RPT)],
                    acc_sh.at[pl.ds(s * _RPT, _RPT)])
    plsc.subcore_barrier()

    bufs = (r0, r1, r2, r3)
    sems = (g0, g1, g2, g3)

    def body(g, carry):
        hnds = [
            pltpu.async_copy(
                hs_hbm.at[src_v.at[g * _G + b]], bufs[b], sems[b])
            for b in range(_G)
        ]
        for b in range(_G):
            hnds[b].wait()
            pltpu.sync_copy(bufs[b], acc_sh.at[dst_v.at[g * _G + b]],
                            add=True)
        return carry

    lax.fori_loop(0, _K // _G, body, 0)
    plsc.subcore_barrier()
    pltpu.sync_copy(acc_sh.at[pl.ds(s * _RPT, _RPT)],
                    out_hbm.at[c, pl.ds(s * _RPT, _RPT)])


def _dinv_body(c0_ref, c1_ref, o_ref):
    deg = c0_ref[...] + c1_ref[...] + 1.0
    o_ref[...] = lax.rsqrt(deg)


def _dense1_body(x_ref, w_ref, dinv_ref, o_ref):
    hw = jnp.dot(x_ref[...], w_ref[...], preferred_element_type=jnp.float32)
    o_ref[...] = dinv_ref[...] * hw


def _combine_body(p0_ref, p1_ref, hs_ref, dinv_ref, b_ref, w_ref, o_ref):
    agg = p0_ref[...] + p1_ref[...] + hs_ref[...]
    h = jnp.maximum(dinv_ref[...] * agg + b_ref[...], 0.0)
    o_ref[...] = dinv_ref[...] * jnp.dot(
        h, w_ref[...], preferred_element_type=jnp.float32)


def _final_body(p0_ref, p1_ref, hs_ref, dinv_ref, b_ref, o_ref):
    h = dinv_ref[...] * (p0_ref[...] + p1_ref[...] + hs_ref[...]) + b_ref[...]
    m = jnp.max(h, axis=1, keepdims=True)
    e = jnp.exp(h - m)
    lse = jnp.log(jnp.sum(e, axis=1, keepdims=True)) + m
    o_ref[...] = h - lse


def _tc(body, out_shape, *ins):
    if isinstance(out_shape, tuple) and isinstance(out_shape[0], tuple):
        os = tuple(jax.ShapeDtypeStruct(s, jnp.float32) for s in out_shape)
    else:
        os = jax.ShapeDtypeStruct(out_shape, jnp.float32)
    return pl.pallas_call(body, out_shape=os)(*ins)


def kernel(x, edge_index, W1, b1, W2, b2, W3, b3):
    src = edge_index[0]
    dst = edge_index[1]
    pad = _EPAD - _E
    src_pad = jnp.arange(pad, dtype=jnp.int32) % _N
    src_p = jnp.concatenate(
        [src, src_pad]).reshape(_K, _NW, _CH).transpose(1, 0, 2)
    dst_pad = jnp.full((pad,), _N, jnp.int32)
    dst_p = jnp.concatenate(
        [dst, dst_pad]).reshape(_K, _NW, _CH).transpose(1, 0, 2)
    zero1 = jnp.zeros((_NPAD,), jnp.float32)
    zero2 = jnp.zeros((_NPAD, _H), jnp.float32)

    cnt = _sc_degree(dst_p, zero1)
    dinv2d = _tc(_dinv_body, (_NPAD // 128, 128),
                 cnt[0].reshape(_NPAD // 128, 128),
                 cnt[1].reshape(_NPAD // 128, 128))
    dinv = dinv2d.reshape(_NPAD)[:_N, None]

    hs1 = _tc(_dense1_body, (_N, _H), x, W1, dinv)
    p = _sc_scatter(hs1, src_p, dst_p, zero2)
    hs2 = _tc(_combine_body, (_N, _H),
              p[0, :_N], p[1, :_N], hs1, dinv, b1[None, :], W2)
    p = _sc_scatter(hs2, src_p, dst_p, zero2)
    hs3 = _tc(_combine_body, (_N, _H),
              p[0, :_N], p[1, :_N], hs2, dinv, b2[None, :], W3)
    p = _sc_scatter(hs3, src_p, dst_p, zero2)
    return _tc(_final_body, (_N, _H),
               p[0, :_N], p[1, :_N], hs3, dinv, b3[None, :])


# speedup vs baseline: 2.2188x; 1.0591x over previous
import functools

import jax
import jax.numpy as jnp
from jax import lax
from jax.experimental import pallas as pl
from jax.experimental.pallas import tpu as pltpu
from jax.experimental.pallas import tpu_sc as plsc

_N = 10000
_E = 320000
_H = 32

_NC = 2
_NS = 16
_NW = _NC * _NS
_CH = 128
_G = 8
_K = 80
_EPAD = _NW * _K * _CH
_NPAD = 10240
_RPT = _NPAD // _NS

_mesh = plsc.VectorSubcoreMesh(core_axis_name="c", subcore_axis_name="s")


@functools.partial(
    pl.kernel,
    out_type=jax.ShapeDtypeStruct((_NC, _NPAD), jnp.float32),
    mesh=_mesh,
    scratch_types=[
        pltpu.VMEM((_K, _CH), jnp.int32),
        pltpu.VMEM((_CH,), jnp.float32),
        pltpu.VMEM_SHARED((_NPAD,), jnp.float32),
    ],
)
def _sc_degree(dst_hbm, zero_hbm, out_hbm, dst_v, ones_v, acc_sh):
    c = lax.axis_index("c")
    s = lax.axis_index("s")
    wid = s * _NC + c
    pltpu.sync_copy(dst_hbm.at[wid], dst_v)
    for i in range(_CH // 16):
        ones_v[pl.ds(i * 16, 16)] = jnp.ones((16,), jnp.float32)
    pltpu.sync_copy(zero_hbm.at[pl.ds(s * _RPT, _RPT)],
                    acc_sh.at[pl.ds(s * _RPT, _RPT)])
    plsc.subcore_barrier()

    def body(j, carry):
        pltpu.sync_copy(ones_v, acc_sh.at[dst_v.at[j]], add=True)
        return carry

    lax.fori_loop(0, _K, body, 0)
    plsc.subcore_barrier()
    pltpu.sync_copy(acc_sh.at[pl.ds(s * _RPT, _RPT)],
                    out_hbm.at[c, pl.ds(s * _RPT, _RPT)])


@functools.partial(
    pl.kernel,
    out_type=jax.ShapeDtypeStruct((_NC, _NPAD, _H), jnp.float32),
    mesh=_mesh,
    scratch_types=[
        pltpu.VMEM((_K, _CH), jnp.int32),
        pltpu.VMEM((_K, _CH), jnp.int32),
        pltpu.VMEM((_CH, _H), jnp.float32),
        pltpu.VMEM((_CH, _H), jnp.float32),
        pltpu.VMEM((_CH, _H), jnp.float32),
        pltpu.VMEM((_CH, _H), jnp.float32),
        pltpu.VMEM((_CH, _H), jnp.float32),
        pltpu.VMEM((_CH, _H), jnp.float32),
        pltpu.VMEM((_CH, _H), jnp.float32),
        pltpu.VMEM((_CH, _H), jnp.float32),
        pltpu.VMEM_SHARED((_NPAD, _H), jnp.float32),
        pltpu.SemaphoreType.DMA,
        pltpu.SemaphoreType.DMA,
        pltpu.SemaphoreType.DMA,
        pltpu.SemaphoreType.DMA,
        pltpu.SemaphoreType.DMA,
        pltpu.SemaphoreType.DMA,
        pltpu.SemaphoreType.DMA,
        pltpu.SemaphoreType.DMA,
    ],
    compiler_params=pltpu.CompilerParams(use_tc_tiling_on_sc=False),
)
def _sc_scatter(hs_hbm, src_hbm, dst_hbm, zero_hbm, out_hbm,
                src_v, dst_v, r0, r1, r2, r3, r4, r5, r6, r7, acc_sh,
                g0, g1, g2, g3, g4, g5, g6, g7):
    c = lax.axis_index("c")
    s = lax.axis_index("s")
    wid = s * _NC + c
    pltpu.sync_copy(src_hbm.at[wid], src_v)
    pltpu.sync_copy(dst_hbm.at[wid], dst_v)
    pltpu.sync_copy(zero_hbm.at[pl.ds(s * _RPT, _RPT)],
                    acc_sh.at[pl.ds(s * _RPT, _RPT)])
    plsc.subcore_barrier()

    bufs = (r0, r1, r2, r3, r4, r5, r6, r7)
    sems = (g0, g1, g2, g3, g4, g5, g6, g7)

    def body(g, carry):
        hnds = [
            pltpu.async_copy(
                hs_hbm.at[src_v.at[g * _G + b]], bufs[b], sems[b])
            for b in range(_G)
        ]
        for b in range(_G):
            hnds[b].wait()
            pltpu.sync_copy(bufs[b], acc_sh.at[dst_v.at[g * _G + b]],
                            add=True)
        return carry

    lax.fori_loop(0, _K // _G, body, 0)
    plsc.subcore_barrier()
    pltpu.sync_copy(acc_sh.at[pl.ds(s * _RPT, _RPT)],
                    out_hbm.at[c, pl.ds(s * _RPT, _RPT)])


def _dinv_body(c0_ref, c1_ref, o_ref):
    deg = c0_ref[...] + c1_ref[...] + 1.0
    o_ref[...] = lax.rsqrt(deg)


def _dense1_body(x_ref, w_ref, dinv_ref, o_ref):
    hw = jnp.dot(x_ref[...], w_ref[...], preferred_element_type=jnp.float32)
    o_ref[...] = dinv_ref[...] * hw


def _combine_body(p0_ref, p1_ref, hs_ref, dinv_ref, b_ref, w_ref, o_ref):
    agg = p0_ref[...] + p1_ref[...] + hs_ref[...]
    h = jnp.maximum(dinv_ref[...] * agg + b_ref[...], 0.0)
    o_ref[...] = dinv_ref[...] * jnp.dot(
        h, w_ref[...], preferred_element_type=jnp.float32)


def _final_body(p0_ref, p1_ref, hs_ref, dinv_ref, b_ref, o_ref):
    h = dinv_ref[...] * (p0_ref[...] + p1_ref[...] + hs_ref[...]) + b_ref[...]
    m = jnp.max(h, axis=1, keepdims=True)
    e = jnp.exp(h - m)
    lse = jnp.log(jnp.sum(e, axis=1, keepdims=True)) + m
    o_ref[...] = h - lse


def _tc(body, out_shape, *ins):
    if isinstance(out_shape, tuple) and isinstance(out_shape[0], tuple):
        os = tuple(jax.ShapeDtypeStruct(s, jnp.float32) for s in out_shape)
    else:
        os = jax.ShapeDtypeStruct(out_shape, jnp.float32)
    return pl.pallas_call(body, out_shape=os)(*ins)


def kernel(x, edge_index, W1, b1, W2, b2, W3, b3):
    src = edge_index[0]
    dst = edge_index[1]
    pad = _EPAD - _E
    src_pad = jnp.arange(pad, dtype=jnp.int32) % _N
    src_p = jnp.concatenate(
        [src, src_pad]).reshape(_K, _NW, _CH).transpose(1, 0, 2)
    dst_pad = jnp.full((pad,), _N, jnp.int32)
    dst_p = jnp.concatenate(
        [dst, dst_pad]).reshape(_K, _NW, _CH).transpose(1, 0, 2)
    zero1 = jnp.zeros((_NPAD,), jnp.float32)
    zero2 = jnp.zeros((_NPAD, _H), jnp.float32)

    cnt = _sc_degree(dst_p, zero1)
    dinv2d = _tc(_dinv_body, (_NPAD // 128, 128),
                 cnt[0].reshape(_NPAD // 128, 128),
                 cnt[1].reshape(_NPAD // 128, 128))
    dinv = dinv2d.reshape(_NPAD)[:_N, None]

    hs1 = _tc(_dense1_body, (_N, _H), x, W1, dinv)
    p = _sc_scatter(hs1, src_p, dst_p, zero2)
    hs2 = _tc(_combine_body, (_N, _H),
              p[0, :_N], p[1, :_N], hs1, dinv, b1[None, :], W2)
    p = _sc_scatter(hs2, src_p, dst_p, zero2)
    hs3 = _tc(_combine_body, (_N, _H),
              p[0, :_N], p[1, :_N], hs2, dinv, b2[None, :], W3)
    p = _sc_scatter(hs3, src_p, dst_p, zero2)
    return _tc(_final_body, (_N, _H),
               p[0, :_N], p[1, :_N], hs3, dinv, b3[None, :])


# speedup vs baseline: 2.3781x; 1.0718x over previous
import functools

import jax
import jax.numpy as jnp
from jax import lax
from jax.experimental import pallas as pl
from jax.experimental.pallas import tpu as pltpu
from jax.experimental.pallas import tpu_sc as plsc

_N = 10000
_E = 320000
_H = 32

_NC = 2
_NS = 16
_NW = _NC * _NS
_CH = 128
_G = 8
_K = 80
_EPAD = _NW * _K * _CH
_NPAD = 10240
_RPT = _NPAD // _NS

_mesh = plsc.VectorSubcoreMesh(core_axis_name="c", subcore_axis_name="s")


@functools.partial(
    pl.kernel,
    out_type=jax.ShapeDtypeStruct((_NC, _NPAD), jnp.float32),
    mesh=_mesh,
    scratch_types=[
        pltpu.VMEM((_K, _CH), jnp.int32),
        pltpu.VMEM((_CH,), jnp.float32),
        pltpu.VMEM_SHARED((_NPAD,), jnp.float32),
    ],
)
def _sc_degree(dst_hbm, zero_hbm, out_hbm, dst_v, ones_v, acc_sh):
    c = lax.axis_index("c")
    s = lax.axis_index("s")
    wid = s * _NC + c
    pltpu.sync_copy(dst_hbm.at[wid], dst_v)
    for i in range(_CH // 16):
        ones_v[pl.ds(i * 16, 16)] = jnp.ones((16,), jnp.float32)
    pltpu.sync_copy(zero_hbm.at[pl.ds(s * _RPT, _RPT)],
                    acc_sh.at[pl.ds(s * _RPT, _RPT)])
    plsc.subcore_barrier()

    def body(j, carry):
        pltpu.sync_copy(ones_v, acc_sh.at[dst_v.at[j]], add=True)
        return carry

    lax.fori_loop(0, _K, body, 0)
    plsc.subcore_barrier()
    pltpu.sync_copy(acc_sh.at[pl.ds(s * _RPT, _RPT)],
                    out_hbm.at[c, pl.ds(s * _RPT, _RPT)])


@functools.partial(
    pl.kernel,
    out_type=jax.ShapeDtypeStruct((_NC, _NPAD, _H), jnp.float32),
    mesh=_mesh,
    scratch_types=[
        pltpu.VMEM((_K, _CH), jnp.int32),
        pltpu.VMEM((_K, _CH), jnp.int32),
        pltpu.VMEM((_CH, _H), jnp.float32),
        pltpu.VMEM((_CH, _H), jnp.float32),
        pltpu.VMEM((_CH, _H), jnp.float32),
        pltpu.VMEM((_CH, _H), jnp.float32),
        pltpu.VMEM((_CH, _H), jnp.float32),
        pltpu.VMEM((_CH, _H), jnp.float32),
        pltpu.VMEM((_CH, _H), jnp.float32),
        pltpu.VMEM((_CH, _H), jnp.float32),
        pltpu.VMEM_SHARED((_NPAD, _H), jnp.float32),
        pltpu.SemaphoreType.DMA,
        pltpu.SemaphoreType.DMA,
        pltpu.SemaphoreType.DMA,
        pltpu.SemaphoreType.DMA,
        pltpu.SemaphoreType.DMA,
        pltpu.SemaphoreType.DMA,
        pltpu.SemaphoreType.DMA,
        pltpu.SemaphoreType.DMA,
    ],
    compiler_params=pltpu.CompilerParams(use_tc_tiling_on_sc=False),
)
def _sc_scatter(hs_hbm, src_hbm, dst_hbm, zero_hbm, out_hbm,
                src_v, dst_v, r0, r1, r2, r3, r4, r5, r6, r7, acc_sh,
                g0, g1, g2, g3, g4, g5, g6, g7):
    c = lax.axis_index("c")
    s = lax.axis_index("s")
    wid = s * _NC + c
    pltpu.sync_copy(src_hbm.at[wid], src_v)
    pltpu.sync_copy(dst_hbm.at[wid], dst_v)
    pltpu.sync_copy(zero_hbm.at[pl.ds(s * _RPT, _RPT)],
                    acc_sh.at[pl.ds(s * _RPT, _RPT)])
    plsc.subcore_barrier()

    bufs = (r0, r1, r2, r3, r4, r5, r6, r7)
    sems = (g0, g1, g2, g3, g4, g5, g6, g7)

    def body(g, carry):
        hnds = [
            pltpu.async_copy(
                hs_hbm.at[src_v.at[g * _G + b]], bufs[b], sems[b])
            for b in range(_G)
        ]
        for b in range(_G):
            hnds[b].wait()
            pltpu.sync_copy(bufs[b], acc_sh.at[dst_v.at[g * _G + b]],
                            add=True)
        return carry

    lax.fori_loop(0, _K // _G, body, 0)
    plsc.subcore_barrier()
    pltpu.sync_copy(acc_sh.at[pl.ds(s * _RPT, _RPT)],
                    out_hbm.at[c, pl.ds(s * _RPT, _RPT)])


def _dinv_body(c0_ref, c1_ref, o_ref):
    deg = c0_ref[...] + c1_ref[...] + 1.0
    o_ref[...] = lax.rsqrt(deg)


def _dense1_body(x_ref, w_ref, dinv_ref, o_ref):
    hw = jnp.dot(x_ref[...], w_ref[...], preferred_element_type=jnp.float32)
    o_ref[...] = dinv_ref[...] * hw


def _combine_body(p_ref, hs_ref, dinv_ref, b_ref, w_ref, o_ref):
    agg = p_ref[0, :_N] + p_ref[1, :_N] + hs_ref[...]
    h = jnp.maximum(dinv_ref[...] * agg + b_ref[...], 0.0)
    o_ref[...] = dinv_ref[...] * jnp.dot(
        h, w_ref[...], preferred_element_type=jnp.float32)


def _final_body(p_ref, hs_ref, dinv_ref, b_ref, o_ref):
    h = dinv_ref[...] * (p_ref[0, :_N] + p_ref[1, :_N] + hs_ref[...]) + b_ref[...]
    m = jnp.max(h, axis=1, keepdims=True)
    e = jnp.exp(h - m)
    lse = jnp.log(jnp.sum(e, axis=1, keepdims=True)) + m
    o_ref[...] = h - lse


def _tc(body, out_shape, *ins):
    if isinstance(out_shape, tuple) and isinstance(out_shape[0], tuple):
        os = tuple(jax.ShapeDtypeStruct(s, jnp.float32) for s in out_shape)
    else:
        os = jax.ShapeDtypeStruct(out_shape, jnp.float32)
    return pl.pallas_call(body, out_shape=os)(*ins)


def kernel(x, edge_index, W1, b1, W2, b2, W3, b3):
    src = edge_index[0]
    dst = edge_index[1]
    pad = _EPAD - _E
    src_pad = jnp.arange(pad, dtype=jnp.int32) % _N
    src_p = jnp.concatenate(
        [src, src_pad]).reshape(_K, _NW, _CH).transpose(1, 0, 2)
    dst_pad = jnp.full((pad,), _N, jnp.int32)
    dst_p = jnp.concatenate(
        [dst, dst_pad]).reshape(_K, _NW, _CH).transpose(1, 0, 2)
    zero1 = jnp.zeros((_NPAD,), jnp.float32)
    zero2 = jnp.zeros((_NPAD, _H), jnp.float32)

    cnt = _sc_degree(dst_p, zero1)
    dinv2d = _tc(_dinv_body, (_NPAD // 128, 128),
                 cnt[0].reshape(_NPAD // 128, 128),
                 cnt[1].reshape(_NPAD // 128, 128))
    dinv = dinv2d.reshape(_NPAD)[:_N, None]

    hs1 = _tc(_dense1_body, (_N, _H), x, W1, dinv)
    p = _sc_scatter(hs1, src_p, dst_p, zero2)
    hs2 = _tc(_combine_body, (_N, _H),
              p, hs1, dinv, b1[None, :], W2)
    p = _sc_scatter(hs2, src_p, dst_p, zero2)
    hs3 = _tc(_combine_body, (_N, _H),
              p, hs2, dinv, b2[None, :], W3)
    p = _sc_scatter(hs3, src_p, dst_p, zero2)
    return _tc(_final_body, (_N, _H),
               p, hs3, dinv, b3[None, :])


# speedup vs baseline: 2.7481x; 1.1556x over previous
import functools

import jax
import jax.numpy as jnp
from jax import lax
from jax.experimental import pallas as pl
from jax.experimental.pallas import tpu as pltpu
from jax.experimental.pallas import tpu_sc as plsc

_N = 10000
_E = 320000
_H = 32

_NC = 2
_NS = 16
_NW = _NC * _NS
_CH = 128
_G = 8
_K = 80
_EPAD = _NW * _K * _CH
_NPAD = 10240
_RPT = _NPAD // _NS

_mesh = plsc.VectorSubcoreMesh(core_axis_name="c", subcore_axis_name="s")


@functools.partial(
    pl.kernel,
    out_type=jax.ShapeDtypeStruct((_NC, _NPAD), jnp.float32),
    mesh=_mesh,
    scratch_types=[
        pltpu.VMEM((_K, _CH), jnp.int32),
        pltpu.VMEM((_CH,), jnp.float32),
        pltpu.VMEM_SHARED((_NPAD,), jnp.float32),
    ],
)
def _sc_degree(dst_hbm, zero_hbm, out_hbm, dst_v, ones_v, acc_sh):
    c = lax.axis_index("c")
    s = lax.axis_index("s")
    wid = s * _NC + c
    pltpu.sync_copy(dst_hbm.at[wid], dst_v)
    for i in range(_CH // 16):
        ones_v[pl.ds(i * 16, 16)] = jnp.ones((16,), jnp.float32)
    pltpu.sync_copy(zero_hbm.at[pl.ds(s * _RPT, _RPT)],
                    acc_sh.at[pl.ds(s * _RPT, _RPT)])
    plsc.subcore_barrier()

    def body(j, carry):
        pltpu.sync_copy(ones_v, acc_sh.at[dst_v.at[j]], add=True)
        return carry

    lax.fori_loop(0, _K, body, 0)
    plsc.subcore_barrier()
    pltpu.sync_copy(acc_sh.at[pl.ds(s * _RPT, _RPT)],
                    out_hbm.at[c, pl.ds(s * _RPT, _RPT)])


@functools.partial(
    pl.kernel,
    out_type=jax.ShapeDtypeStruct((_NC, _NPAD, _H), jnp.float32),
    mesh=_mesh,
    scratch_types=[
        pltpu.VMEM((_K, _CH), jnp.int32),
        pltpu.VMEM((_K, _CH), jnp.int32),
        pltpu.VMEM((_CH, _H), jnp.float32),
        pltpu.VMEM((_CH, _H), jnp.float32),
        pltpu.VMEM((_CH, _H), jnp.float32),
        pltpu.VMEM((_CH, _H), jnp.float32),
        pltpu.VMEM((_CH, _H), jnp.float32),
        pltpu.VMEM((_CH, _H), jnp.float32),
        pltpu.VMEM((_CH, _H), jnp.float32),
        pltpu.VMEM((_CH, _H), jnp.float32),
        pltpu.VMEM_SHARED((_NPAD, _H), jnp.float32),
        pltpu.SemaphoreType.DMA,
        pltpu.SemaphoreType.DMA,
        pltpu.SemaphoreType.DMA,
        pltpu.SemaphoreType.DMA,
        pltpu.SemaphoreType.DMA,
        pltpu.SemaphoreType.DMA,
        pltpu.SemaphoreType.DMA,
        pltpu.SemaphoreType.DMA,
    ],
    compiler_params=pltpu.CompilerParams(use_tc_tiling_on_sc=False),
)
def _sc_scatter(hs_hbm, src_hbm, dst_hbm, zero_hbm, out_hbm,
                src_v, dst_v, r0, r1, r2, r3, r4, r5, r6, r7, acc_sh,
                g0, g1, g2, g3, g4, g5, g6, g7):
    c = lax.axis_index("c")
    s = lax.axis_index("s")
    wid = s * _NC + c
    pltpu.sync_copy(src_hbm.at[wid], src_v)
    pltpu.sync_copy(dst_hbm.at[wid], dst_v)
    pltpu.sync_copy(zero_hbm.at[pl.ds(s * _RPT, _RPT)],
                    acc_sh.at[pl.ds(s * _RPT, _RPT)])
    plsc.subcore_barrier()

    bufs = (r0, r1, r2, r3, r4, r5, r6, r7)
    sems = (g0, g1, g2, g3, g4, g5, g6, g7)

    def body(g, carry):
        hnds = [
            pltpu.async_copy(
                hs_hbm.at[src_v.at[g * _G + b]], bufs[b], sems[b])
            for b in range(_G)
        ]
        for b in range(_G):
            hnds[b].wait()
            pltpu.sync_copy(bufs[b], acc_sh.at[dst_v.at[g * _G + b]],
                            add=True)
        return carry

    lax.fori_loop(0, _K // _G, body, 0)
    plsc.subcore_barrier()
    pltpu.sync_copy(acc_sh.at[pl.ds(s * _RPT, _RPT)],
                    out_hbm.at[c, pl.ds(s * _RPT, _RPT)])


def _dinv_body(c0_ref, c1_ref, o_ref):
    deg = c0_ref[...] + c1_ref[...] + 1.0
    o_ref[...] = lax.rsqrt(deg)


_PK = _N * _H // 128
_PKPAD = _NPAD * _H // 128


def _dense1_body(x_ref, w_ref, dinv_ref, o_ref):
    hw = jnp.dot(x_ref[...], w_ref[...], preferred_element_type=jnp.float32)
    o_ref[...] = dinv_ref[...] * hw


def _combine_body(p_ref, hs_ref, dinv_ref, b_ref, w_ref, o_ref):
    agg = p_ref[0, :_PK] + p_ref[1, :_PK] + hs_ref[...]
    h = jnp.maximum(dinv_ref[...] * agg + b_ref[...], 0.0)
    o_ref[...] = dinv_ref[...] * jnp.dot(
        h, w_ref[...], preferred_element_type=jnp.float32)


def _final_body(p_ref, hs_ref, dinv_ref, b_ref, o_ref):
    h = dinv_ref[...] * (p_ref[0, :_N] + p_ref[1, :_N] + hs_ref[...]) + b_ref[...]
    m = jnp.max(h, axis=1, keepdims=True)
    e = jnp.exp(h - m)
    lse = jnp.log(jnp.sum(e, axis=1, keepdims=True)) + m
    o_ref[...] = h - lse


def _tc(body, out_shape, *ins):
    if isinstance(out_shape, tuple) and isinstance(out_shape[0], tuple):
        os = tuple(jax.ShapeDtypeStruct(s, jnp.float32) for s in out_shape)
    else:
        os = jax.ShapeDtypeStruct(out_shape, jnp.float32)
    return pl.pallas_call(body, out_shape=os)(*ins)


def kernel(x, edge_index, W1, b1, W2, b2, W3, b3):
    src = edge_index[0]
    dst = edge_index[1]
    pad = _EPAD - _E
    src_pad = jnp.arange(pad, dtype=jnp.int32) % _N
    src_p = jnp.concatenate(
        [src, src_pad]).reshape(_K, _NW, _CH).transpose(1, 0, 2)
    dst_pad = jnp.full((pad,), _N, jnp.int32)
    dst_p = jnp.concatenate(
        [dst, dst_pad]).reshape(_K, _NW, _CH).transpose(1, 0, 2)
    zero1 = jnp.zeros((_NPAD,), jnp.float32)
    zero2 = jnp.zeros((_NPAD, _H), jnp.float32)

    cnt = _sc_degree(dst_p, zero1)
    dinv2d = _tc(_dinv_body, (_NPAD // 128, 128),
                 cnt[0].reshape(_NPAD // 128, 128),
                 cnt[1].reshape(_NPAD // 128, 128))
    dinv_flat = dinv2d.reshape(_NPAD)
    dinv = dinv_flat[:_N, None]
    dinv_pk = jnp.repeat(dinv_flat, _H).reshape(_PKPAD, 128)[:_PK]
    blk = jax.scipy.linalg.block_diag
    rep = 128 // _H
    W2b = blk(*([W2] * rep))
    W3b = blk(*([W3] * rep))
    bt1 = jnp.tile(b1, rep)[None, :]
    bt2 = jnp.tile(b2, rep)[None, :]

    hs1 = _tc(_dense1_body, (_N, _H), x, W1, dinv)
    p = _sc_scatter(hs1, src_p, dst_p, zero2)
    hs2 = _tc(_combine_body, (_PK, 128),
              p.reshape(2, _PKPAD, 128), hs1.reshape(_PK, 128), dinv_pk,
              bt1, W2b)
    p = _sc_scatter(hs2.reshape(_N, _H), src_p, dst_p, zero2)
    hs3 = _tc(_combine_body, (_PK, 128),
              p.reshape(2, _PKPAD, 128), hs2, dinv_pk, bt2, W3b)
    p = _sc_scatter(hs3.reshape(_N, _H), src_p, dst_p, zero2)
    return _tc(_final_body, (_N, _H),
               p, hs3.reshape(_N, _H), dinv, b3[None, :])


# speedup vs baseline: 2.8751x; 1.0462x over previous
import functools

import jax
import jax.numpy as jnp
from jax import lax
from jax.experimental import pallas as pl
from jax.experimental.pallas import tpu as pltpu
from jax.experimental.pallas import tpu_sc as plsc

_N = 10000
_E = 320000
_H = 32

_NC = 2
_NS = 16
_NW = _NC * _NS
_CH = 128
_G = 8
_K = 80
_EPAD = _NW * _K * _CH
_NPAD = 10240
_RPT = _NPAD // _NS

_mesh = plsc.VectorSubcoreMesh(core_axis_name="c", subcore_axis_name="s")


@functools.partial(
    pl.kernel,
    out_type=jax.ShapeDtypeStruct((_NC, _NPAD), jnp.float32),
    mesh=_mesh,
    scratch_types=[
        pltpu.VMEM((_K, _CH), jnp.int32),
        pltpu.VMEM((_CH,), jnp.float32),
        pltpu.VMEM_SHARED((_NPAD,), jnp.float32),
    ],
)
def _sc_degree(dst_hbm, zero_hbm, out_hbm, dst_v, ones_v, acc_sh):
    c = lax.axis_index("c")
    s = lax.axis_index("s")
    wid = s * _NC + c
    pltpu.sync_copy(dst_hbm.at[wid], dst_v)
    for i in range(_CH // 16):
        ones_v[pl.ds(i * 16, 16)] = jnp.ones((16,), jnp.float32)
    pltpu.sync_copy(zero_hbm.at[pl.ds(s * _RPT, _RPT)],
                    acc_sh.at[pl.ds(s * _RPT, _RPT)])
    plsc.subcore_barrier()

    def body(j, carry):
        pltpu.sync_copy(ones_v, acc_sh.at[dst_v.at[j]], add=True)
        return carry

    lax.fori_loop(0, _K, body, 0)
    plsc.subcore_barrier()
    pltpu.sync_copy(acc_sh.at[pl.ds(s * _RPT, _RPT)],
                    out_hbm.at[c, pl.ds(s * _RPT, _RPT)])


@functools.partial(
    pl.kernel,
    out_type=jax.ShapeDtypeStruct((_NC, _NPAD, _H), jnp.float32),
    mesh=_mesh,
    scratch_types=[
        pltpu.VMEM((_K, _CH), jnp.int32),
        pltpu.VMEM((_K, _CH), jnp.int32),
        pltpu.VMEM((_CH, _H), jnp.float32),
        pltpu.VMEM((_CH, _H), jnp.float32),
        pltpu.VMEM((_CH, _H), jnp.float32),
        pltpu.VMEM((_CH, _H), jnp.float32),
        pltpu.VMEM((_CH, _H), jnp.float32),
        pltpu.VMEM((_CH, _H), jnp.float32),
        pltpu.VMEM((_CH, _H), jnp.float32),
        pltpu.VMEM((_CH, _H), jnp.float32),
        pltpu.VMEM_SHARED((_NPAD, _H), jnp.float32),
        pltpu.SemaphoreType.DMA,
        pltpu.SemaphoreType.DMA,
        pltpu.SemaphoreType.DMA,
        pltpu.SemaphoreType.DMA,
        pltpu.SemaphoreType.DMA,
        pltpu.SemaphoreType.DMA,
        pltpu.SemaphoreType.DMA,
        pltpu.SemaphoreType.DMA,
        pltpu.SemaphoreType.DMA,
        pltpu.SemaphoreType.DMA,
        pltpu.SemaphoreType.DMA,
        pltpu.SemaphoreType.DMA,
        pltpu.SemaphoreType.DMA,
        pltpu.SemaphoreType.DMA,
        pltpu.SemaphoreType.DMA,
        pltpu.SemaphoreType.DMA,
    ],
    compiler_params=pltpu.CompilerParams(use_tc_tiling_on_sc=False),
)
def _sc_scatter(hs_hbm, src_hbm, dst_hbm, zero_hbm, out_hbm,
                src_v, dst_v, r0, r1, r2, r3, r4, r5, r6, r7, acc_sh,
                g0, g1, g2, g3, g4, g5, g6, g7,
                t0, t1, t2, t3, t4, t5, t6, t7):
    c = lax.axis_index("c")
    s = lax.axis_index("s")
    wid = s * _NC + c
    pltpu.sync_copy(src_hbm.at[wid], src_v)
    pltpu.sync_copy(dst_hbm.at[wid], dst_v)
    pltpu.sync_copy(zero_hbm.at[pl.ds(s * _RPT, _RPT)],
                    acc_sh.at[pl.ds(s * _RPT, _RPT)])
    plsc.subcore_barrier()

    bufs = (r0, r1, r2, r3, r4, r5, r6, r7)
    sems = (g0, g1, g2, g3, g4, g5, g6, g7)
    ssems = (t0, t1, t2, t3, t4, t5, t6, t7)

    def body(g, carry):
        hnds = [
            pltpu.async_copy(
                hs_hbm.at[src_v.at[g * _G + b]], bufs[b], sems[b])
            for b in range(_G)
        ]
        shnds = []
        for b in range(_G):
            hnds[b].wait()
            shnds.append(pltpu.async_copy(
                bufs[b], acc_sh.at[dst_v.at[g * _G + b]], ssems[b],
                add=True))
        for b in range(_G):
            shnds[b].wait()
        return carry

    lax.fori_loop(0, _K // _G, body, 0)
    plsc.subcore_barrier()
    pltpu.sync_copy(acc_sh.at[pl.ds(s * _RPT, _RPT)],
                    out_hbm.at[c, pl.ds(s * _RPT, _RPT)])


def _dinv_body(c0_ref, c1_ref, o_ref):
    deg = c0_ref[...] + c1_ref[...] + 1.0
    o_ref[...] = lax.rsqrt(deg)


_PK = _N * _H // 128
_PKPAD = _NPAD * _H // 128


def _dense1_body(x_ref, w_ref, dinv_ref, o_ref):
    hw = jnp.dot(x_ref[...], w_ref[...], preferred_element_type=jnp.float32)
    o_ref[...] = dinv_ref[...] * hw


def _combine_body(p_ref, hs_ref, dinv_ref, b_ref, w_ref, o_ref):
    agg = p_ref[0, :_PK] + p_ref[1, :_PK] + hs_ref[...]
    h = jnp.maximum(dinv_ref[...] * agg + b_ref[...], 0.0)
    o_ref[...] = dinv_ref[...] * jnp.dot(
        h, w_ref[...], preferred_element_type=jnp.float32)


def _final_body(p_ref, hs_ref, dinv_ref, b_ref, o_ref):
    h = dinv_ref[...] * (p_ref[0, :_N] + p_ref[1, :_N] + hs_ref[...]) + b_ref[...]
    m = jnp.max(h, axis=1, keepdims=True)
    e = jnp.exp(h - m)
    lse = jnp.log(jnp.sum(e, axis=1, keepdims=True)) + m
    o_ref[...] = h - lse


def _tc(body, out_shape, *ins):
    if isinstance(out_shape, tuple) and isinstance(out_shape[0], tuple):
        os = tuple(jax.ShapeDtypeStruct(s, jnp.float32) for s in out_shape)
    else:
        os = jax.ShapeDtypeStruct(out_shape, jnp.float32)
    return pl.pallas_call(body, out_shape=os)(*ins)


def kernel(x, edge_index, W1, b1, W2, b2, W3, b3):
    src = edge_index[0]
    dst = edge_index[1]
    pad = _EPAD - _E
    src_pad = jnp.arange(pad, dtype=jnp.int32) % _N
    src_p = jnp.concatenate(
        [src, src_pad]).reshape(_K, _NW, _CH).transpose(1, 0, 2)
    dst_pad = jnp.full((pad,), _N, jnp.int32)
    dst_p = jnp.concatenate(
        [dst, dst_pad]).reshape(_K, _NW, _CH).transpose(1, 0, 2)
    zero1 = jnp.zeros((_NPAD,), jnp.float32)
    zero2 = jnp.zeros((_NPAD, _H), jnp.float32)

    cnt = _sc_degree(dst_p, zero1)
    dinv2d = _tc(_dinv_body, (_NPAD // 128, 128),
                 cnt[0].reshape(_NPAD // 128, 128),
                 cnt[1].reshape(_NPAD // 128, 128))
    dinv_flat = dinv2d.reshape(_NPAD)
    dinv = dinv_flat[:_N, None]
    dinv_pk = jnp.repeat(dinv_flat, _H).reshape(_PKPAD, 128)[:_PK]
    blk = jax.scipy.linalg.block_diag
    rep = 128 // _H
    W2b = blk(*([W2] * rep))
    W3b = blk(*([W3] * rep))
    bt1 = jnp.tile(b1, rep)[None, :]
    bt2 = jnp.tile(b2, rep)[None, :]

    hs1 = _tc(_dense1_body, (_N, _H), x, W1, dinv)
    p = _sc_scatter(hs1, src_p, dst_p, zero2)
    hs2 = _tc(_combine_body, (_PK, 128),
              p.reshape(2, _PKPAD, 128), hs1.reshape(_PK, 128), dinv_pk,
              bt1, W2b)
    p = _sc_scatter(hs2.reshape(_N, _H), src_p, dst_p, zero2)
    hs3 = _tc(_combine_body, (_PK, 128),
              p.reshape(2, _PKPAD, 128), hs2, dinv_pk, bt2, W3b)
    p = _sc_scatter(hs3.reshape(_N, _H), src_p, dst_p, zero2)
    return _tc(_final_body, (_N, _H),
               p, hs3.reshape(_N, _H), dinv, b3[None, :])
